# Initial kernel scaffold; baseline (speedup 1.0000x reference)
import functools
import jax
import jax.numpy as jnp
from jax import lax
from jax.experimental import pallas as pl
from jax.experimental.pallas import tpu as pltpu, tpu_sc as plsc

_N = 10000
_E = 320000
_D = 128
_H = 128
_C = 8

_NC = 2
_NS = 16
_NW = _NC * _NS
_CH = 128
_NCHUNK = 80
_EPT = _CH * _NCHUNK
_EPAD = _EPT * _NW
_RPT = 626
_NPAD = _RPT * _NS

_mesh = plsc.VectorSubcoreMesh(core_axis_name="c", subcore_axis_name="s")


@functools.partial(
    pl.kernel,
    out_type=jax.ShapeDtypeStruct((_NC, _NPAD, 8), jnp.float32),
    mesh=_mesh,
    scratch_types=[
        pltpu.VMEM((_NCHUNK, _CH), jnp.int32),
        pltpu.VMEM((_CH, 8), jnp.float32),
        pltpu.VMEM_SHARED((_NPAD, 8), jnp.float32),
    ],
)
def _deg_kernel(dst_hbm, ones_hbm, zeros_hbm, out_hbm, idx_v, ones_v, acc_sh):
    cid = lax.axis_index("c")
    sid = lax.axis_index("s")
    wid = cid * _NS + sid
    pltpu.sync_copy(dst_hbm.at[wid], idx_v)
    pltpu.sync_copy(ones_hbm, ones_v)
    pltpu.sync_copy(zeros_hbm.at[pl.ds(sid * _RPT, _RPT)],
                    acc_sh.at[pl.ds(sid * _RPT, _RPT)])
    plsc.subcore_barrier()

    def step(j, carry):
        pltpu.sync_copy(ones_v, acc_sh.at[idx_v.at[j]], add=True)
        return carry

    lax.fori_loop(0, _NCHUNK, step, 0)
    plsc.subcore_barrier()
    pltpu.sync_copy(acc_sh.at[pl.ds(sid * _RPT, _RPT)],
                    out_hbm.at[cid].at[pl.ds(sid * _RPT, _RPT)])


def _make_agg_kernel(hd):
    @functools.partial(
        pl.kernel,
        out_type=jax.ShapeDtypeStruct((_NC, _NPAD, hd), jnp.float32),
        mesh=_mesh,
        scratch_types=[
            pltpu.VMEM((_NCHUNK, _CH), jnp.int32),
            pltpu.VMEM((_NCHUNK, _CH), jnp.int32),
            pltpu.VMEM((_CH, hd), jnp.float32),
            pltpu.VMEM_SHARED((_NPAD, hd), jnp.float32),
            pltpu.SemaphoreType.DMA,
        ],
    )
    def agg(h_hbm, src_hbm, dst_hbm, zeros_hbm, out_hbm,
            src_v, dst_v, rows_v, acc_sh, sem):
        cid = lax.axis_index("c")
        sid = lax.axis_index("s")
        wid = cid * _NS + sid
        pltpu.sync_copy(src_hbm.at[wid], src_v)
        pltpu.sync_copy(dst_hbm.at[wid], dst_v)
        pltpu.sync_copy(zeros_hbm.at[pl.ds(sid * _RPT, _RPT)],
                        acc_sh.at[pl.ds(sid * _RPT, _RPT)])
        plsc.subcore_barrier()

        def step(j, carry):
            pltpu.async_copy(h_hbm.at[src_v.at[j]], rows_v, sem).wait()
            pltpu.sync_copy(rows_v, acc_sh.at[dst_v.at[j]], add=True)
            return carry

        lax.fori_loop(0, _NCHUNK, step, 0)
        plsc.subcore_barrier()
        pltpu.sync_copy(acc_sh.at[pl.ds(sid * _RPT, _RPT)],
                        out_hbm.at[cid].at[pl.ds(sid * _RPT, _RPT)])

    return agg


_agg128 = _make_agg_kernel(_H)
_agg64 = _make_agg_kernel(_H // 2)


_BLK = 1000


def _dinv_of(deg_ref):
    deg = deg_ref[0, :, 0] + deg_ref[1, :, 0] + 1.0
    return lax.rsqrt(jnp.maximum(deg, 1.0))


def _mm1_body(deg_ref, x_ref, w_ref, o_ref):
    dinv = _dinv_of(deg_ref)
    o_ref[...] = (x_ref[...] @ w_ref[...]) * dinv[:, None]


def _mm2_body(deg_ref, p_ref, hp_ref, b_ref, w_ref, o_ref):
    dinv = _dinv_of(deg_ref)
    agg = p_ref[0] + p_ref[1] + hp_ref[...]
    h = jnp.maximum(agg * dinv[:, None] + b_ref[...], 0.0)
    o_ref[...] = (h @ w_ref[...]) * dinv[:, None]


def _mm3_body(deg_ref, p_ref, hp_ref, b_ref, w_ref, bfc_ref, o_ref):
    dinv = _dinv_of(deg_ref)
    agg = p_ref[0] + p_ref[1] + hp_ref[...]
    h = jnp.maximum(agg * dinv[:, None] + b_ref[...], 0.0)
    logits = h @ w_ref[...] + bfc_ref[...]
    m = jnp.max(logits, axis=1, keepdims=True)
    lse = m + jnp.log(jnp.sum(jnp.exp(logits - m), axis=1, keepdims=True))
    o_ref[...] = logits - lse


def _row_block(nd):
    return pl.BlockSpec((_BLK, nd), lambda i: (i, 0))


def _deg_block():
    return pl.BlockSpec((2, _BLK, 8), lambda i: (0, i, 0))


def _part_block(nd):
    return pl.BlockSpec((2, _BLK, nd), lambda i: (0, i, 0))


def _full(shape):
    return pl.BlockSpec(shape, lambda i: tuple(0 for _ in shape))


def kernel(x, edge_index, W1, b1, W2, b2, Wfc, bfc):
    src = edge_index[0]
    dst = edge_index[1]
    npad = _EPAD - _E
    src3 = jnp.concatenate(
        [src, jnp.zeros((npad,), jnp.int32)]).reshape(_NW, _NCHUNK, _CH)
    dst3 = jnp.concatenate(
        [dst, jnp.full((npad,), _N, jnp.int32)]).reshape(_NW, _NCHUNK, _CH)

    ones8 = jnp.ones((_CH, 8), jnp.float32)
    z8 = jnp.zeros((_NPAD, 8), jnp.float32)
    z128 = jnp.zeros((_NPAD, _H), jnp.float32)
    z64 = jnp.zeros((_NPAD, _H // 2), jnp.float32)

    degp = _deg_kernel(dst3, ones8, z8)[:, :_N, :]

    grid = (_N // _BLK,)
    h1p = pl.pallas_call(
        _mm1_body,
        grid=grid,
        in_specs=[_deg_block(), _row_block(_D), _full((_D, _H))],
        out_specs=_row_block(_H),
        out_shape=jax.ShapeDtypeStruct((_N, _H), jnp.float32),
    )(degp, x, W1)

    p1 = _agg128(h1p, src3, dst3, z128)[:, :_N, :]

    h2p = pl.pallas_call(
        _mm2_body,
        grid=grid,
        in_specs=[_deg_block(), _part_block(_H), _row_block(_H),
                  _full((1, _H)), _full((_H, _H // 2))],
        out_specs=_row_block(_H // 2),
        out_shape=jax.ShapeDtypeStruct((_N, _H // 2), jnp.float32),
    )(degp, p1, h1p, b1.reshape(1, _H), W2)

    p2 = _agg64(h2p, src3, dst3, z64)[:, :_N, :]

    out = pl.pallas_call(
        _mm3_body,
        grid=grid,
        in_specs=[_deg_block(), _part_block(_H // 2), _row_block(_H // 2),
                  _full((1, _H // 2)), _full((_H // 2, _C)), _full((1, _C))],
        out_specs=_row_block(_C),
        out_shape=jax.ShapeDtypeStruct((_N, _C), jnp.float32),
    )(degp, p2, h2p, b2.reshape(1, _H // 2), Wfc, bfc.reshape(1, _C))

    return out


# speedup vs baseline: 7.2388x; 7.2388x over previous
import functools
import jax
import jax.numpy as jnp
from jax import lax
from jax.experimental import pallas as pl
from jax.experimental.pallas import tpu as pltpu, tpu_sc as plsc

_N = 10000
_E = 320000
_D = 128
_H = 128
_C = 8

_NC = 2
_NS = 16
_NW = _NC * _NS
_CH = 128
_NCHUNK = 80
_EPT = _CH * _NCHUNK
_EPAD = _EPT * _NW
_DW = 128
_RPT = 632
_NPAD = _RPT * _NS

@functools.lru_cache(maxsize=None)
def _make_deg_kernel():
    mesh = plsc.VectorSubcoreMesh(core_axis_name="c", subcore_axis_name="s")

    @functools.partial(
        pl.kernel,
        out_type=jax.ShapeDtypeStruct((_NC, _NPAD, _DW), jnp.float32),
        mesh=mesh,
        scratch_types=[
            pltpu.VMEM((_NCHUNK, _CH), jnp.int32),
            pltpu.VMEM((_CH, _DW), jnp.float32),
            pltpu.VMEM_SHARED((_NPAD, _DW), jnp.float32),
        ],
    )
    def deg(dst_hbm, ones_hbm, zeros_hbm, out_hbm, idx_v, ones_v, acc_sh):
        cid = lax.axis_index("c")
        sid = lax.axis_index("s")
        wid = cid * _NS + sid
        pltpu.sync_copy(dst_hbm.at[wid], idx_v)
        pltpu.sync_copy(ones_hbm, ones_v)
        pltpu.sync_copy(zeros_hbm.at[pl.ds(sid * _RPT, _RPT)],
                        acc_sh.at[pl.ds(sid * _RPT, _RPT)])
        plsc.subcore_barrier()

        def step(j, carry):
            pltpu.sync_copy(ones_v, acc_sh.at[idx_v.at[j]], add=True)
            return carry

        lax.fori_loop(0, _NCHUNK, step, 0)
        plsc.subcore_barrier()
        pltpu.sync_copy(acc_sh.at[pl.ds(sid * _RPT, _RPT)],
                        out_hbm.at[cid].at[pl.ds(sid * _RPT, _RPT)])

    return deg


@functools.lru_cache(maxsize=None)
def _make_agg_kernel(hd):
    mesh = plsc.VectorSubcoreMesh(core_axis_name="c", subcore_axis_name="s")

    @functools.partial(
        pl.kernel,
        out_type=jax.ShapeDtypeStruct((_NC, _NPAD, hd), jnp.float32),
        mesh=mesh,
        scratch_types=[
            pltpu.VMEM((_NCHUNK, _CH), jnp.int32),
            pltpu.VMEM((_NCHUNK, _CH), jnp.int32),
            pltpu.VMEM((_CH, hd), jnp.float32),
            pltpu.VMEM_SHARED((_NPAD, hd), jnp.float32),
            pltpu.SemaphoreType.DMA,
        ],
    )
    def agg(h_hbm, src_hbm, dst_hbm, zeros_hbm, out_hbm,
            src_v, dst_v, rows_v, acc_sh, sem):
        cid = lax.axis_index("c")
        sid = lax.axis_index("s")
        wid = cid * _NS + sid
        pltpu.sync_copy(src_hbm.at[wid], src_v)
        pltpu.sync_copy(dst_hbm.at[wid], dst_v)
        pltpu.sync_copy(zeros_hbm.at[pl.ds(sid * _RPT, _RPT)],
                        acc_sh.at[pl.ds(sid * _RPT, _RPT)])
        plsc.subcore_barrier()

        def step(j, carry):
            pltpu.async_copy(h_hbm.at[src_v.at[j]], rows_v, sem).wait()
            pltpu.sync_copy(rows_v, acc_sh.at[dst_v.at[j]], add=True)
            return carry

        lax.fori_loop(0, _NCHUNK, step, 0)
        plsc.subcore_barrier()
        pltpu.sync_copy(acc_sh.at[pl.ds(sid * _RPT, _RPT)],
                        out_hbm.at[cid].at[pl.ds(sid * _RPT, _RPT)])

    return agg


_BLK = 1000


def _dinv_of(deg_ref):
    deg = deg_ref[0, :, 0] + deg_ref[1, :, 0] + 1.0
    return lax.rsqrt(jnp.maximum(deg, 1.0))


def _mm1_body(deg_ref, x_ref, w_ref, o_ref):
    dinv = _dinv_of(deg_ref)
    o_ref[...] = (x_ref[...] @ w_ref[...]) * dinv[:, None]


def _mm2_body(deg_ref, p_ref, hp_ref, b_ref, w_ref, o_ref):
    dinv = _dinv_of(deg_ref)
    agg = p_ref[0] + p_ref[1] + hp_ref[...]
    h = jnp.maximum(agg * dinv[:, None] + b_ref[...], 0.0)
    res = (h @ w_ref[...]) * dinv[:, None]
    o_ref[...] = jnp.concatenate([res, jnp.zeros_like(res)], axis=1)


def _mm3_body(deg_ref, p_ref, hp_ref, b_ref, w_ref, bfc_ref, o_ref):
    dinv = _dinv_of(deg_ref)
    agg = (p_ref[0] + p_ref[1] + hp_ref[...])[:, : _H // 2]
    h = jnp.maximum(agg * dinv[:, None] + b_ref[...], 0.0)
    logits = h @ w_ref[...] + bfc_ref[...]
    m = jnp.max(logits, axis=1, keepdims=True)
    lse = m + jnp.log(jnp.sum(jnp.exp(logits - m), axis=1, keepdims=True))
    o_ref[...] = logits - lse


def _row_block(nd):
    return pl.BlockSpec((_BLK, nd), lambda i: (i, 0))


def _deg_block():
    return pl.BlockSpec((2, _BLK, _DW), lambda i: (0, i, 0))


def _part_block(nd):
    return pl.BlockSpec((2, _BLK, nd), lambda i: (0, i, 0))


def _full(shape):
    return pl.BlockSpec(shape, lambda i: tuple(0 for _ in shape))


def kernel(x, edge_index, W1, b1, W2, b2, Wfc, bfc):
    src = edge_index[0]
    dst = edge_index[1]
    npad = _EPAD - _E
    src3 = jnp.concatenate(
        [src, jnp.zeros((npad,), jnp.int32)]).reshape(_NW, _NCHUNK, _CH)
    dst3 = jnp.concatenate(
        [dst, jnp.full((npad,), _N, jnp.int32)]).reshape(_NW, _NCHUNK, _CH)

    ones8 = jnp.ones((_CH, _DW), jnp.float32)
    z8 = jnp.zeros((_NPAD, _DW), jnp.float32)
    z128 = jnp.zeros((_NPAD, _H), jnp.float32)

    degp = _make_deg_kernel()(dst3, ones8, z8)[:, :_N, :]

    grid = (_N // _BLK,)
    h1p = pl.pallas_call(
        _mm1_body,
        grid=grid,
        in_specs=[_deg_block(), _row_block(_D), _full((_D, _H))],
        out_specs=_row_block(_H),
        out_shape=jax.ShapeDtypeStruct((_N, _H), jnp.float32),
    )(degp, x, W1)

    p1 = _make_agg_kernel(_H)(h1p, src3, dst3, z128)[:, :_N, :]

    h2p = pl.pallas_call(
        _mm2_body,
        grid=grid,
        in_specs=[_deg_block(), _part_block(_H), _row_block(_H),
                  _full((1, _H)), _full((_H, _H // 2))],
        out_specs=_row_block(_H),
        out_shape=jax.ShapeDtypeStruct((_N, _H), jnp.float32),
    )(degp, p1, h1p, b1.reshape(1, _H), W2)

    p2 = _make_agg_kernel(_H)(h2p, src3, dst3, z128)[:, :_N, :]

    out = pl.pallas_call(
        _mm3_body,
        grid=grid,
        in_specs=[_deg_block(), _part_block(_H), _row_block(_H),
                  _full((1, _H // 2)), _full((_H // 2, _C)), _full((1, _C))],
        out_specs=_row_block(_C),
        out_shape=jax.ShapeDtypeStruct((_N, _C), jnp.float32),
    )(degp, p2, h2p, b2.reshape(1, _H // 2), Wfc, bfc.reshape(1, _C))

    return out


# speedup vs baseline: 8.3243x; 1.1500x over previous
import functools
import jax
import jax.numpy as jnp
from jax import lax
from jax.experimental import pallas as pl
from jax.experimental.pallas import tpu as pltpu, tpu_sc as plsc

_N = 10000
_E = 320000
_D = 128
_H = 128
_C = 8

_NC = 2
_NS = 16
_NW = _NC * _NS
_CH = 64
_NCHUNK = 160
_IGRP = 16
_EPT = _CH * _NCHUNK
_EPAD = _EPT * _NW
_DW = 128
_RPT = 632
_NPAD = _RPT * _NS

@functools.lru_cache(maxsize=None)
def _make_deg_kernel():
    mesh = plsc.VectorSubcoreMesh(core_axis_name="c", subcore_axis_name="s")

    @functools.partial(
        pl.kernel,
        out_type=jax.ShapeDtypeStruct((_NC, _NPAD, _DW), jnp.float32),
        mesh=mesh,
        scratch_types=[
            pltpu.VMEM((_NCHUNK, _CH), jnp.int32),
            pltpu.VMEM((_CH, _DW), jnp.float32),
            pltpu.VMEM_SHARED((_NPAD, _DW), jnp.float32),
        ],
    )
    def deg(dst_hbm, ones_hbm, zeros_hbm, out_hbm, idx_v, ones_v, acc_sh):
        cid = lax.axis_index("c")
        sid = lax.axis_index("s")
        wid = cid * _NS + sid
        pltpu.sync_copy(dst_hbm.at[wid], idx_v)
        pltpu.sync_copy(ones_hbm, ones_v)
        pltpu.sync_copy(zeros_hbm.at[pl.ds(sid * _RPT, _RPT)],
                        acc_sh.at[pl.ds(sid * _RPT, _RPT)])
        plsc.subcore_barrier()

        def step(j, carry):
            pltpu.sync_copy(ones_v, acc_sh.at[idx_v.at[j]], add=True)
            return carry

        lax.fori_loop(0, _NCHUNK, step, 0)
        plsc.subcore_barrier()
        pltpu.sync_copy(acc_sh.at[pl.ds(sid * _RPT, _RPT)],
                        out_hbm.at[cid].at[pl.ds(sid * _RPT, _RPT)])

    return deg


@functools.lru_cache(maxsize=None)
def _make_agg_kernel(hd):
    mesh = plsc.VectorSubcoreMesh(core_axis_name="c", subcore_axis_name="s")

    @functools.partial(
        pl.kernel,
        out_type=jax.ShapeDtypeStruct((_NC, _NPAD, hd), jnp.float32),
        mesh=mesh,
        scratch_types=[
            pltpu.VMEM((_IGRP, _CH), jnp.int32),
            pltpu.VMEM((_IGRP, _CH), jnp.int32),
            pltpu.VMEM((_CH, hd), jnp.float32),
            pltpu.VMEM((_CH, hd), jnp.float32),
            pltpu.VMEM_SHARED((_NPAD, hd), jnp.float32),
            pltpu.SemaphoreType.DMA,
            pltpu.SemaphoreType.DMA,
        ],
    )
    def agg(h_hbm, src_hbm, dst_hbm, zeros_hbm, out_hbm,
            src_v, dst_v, rows_a, rows_b, acc_sh, sem_a, sem_b):
        cid = lax.axis_index("c")
        sid = lax.axis_index("s")
        wid = cid * _NS + sid
        pltpu.sync_copy(zeros_hbm.at[pl.ds(sid * _RPT, _RPT)],
                        acc_sh.at[pl.ds(sid * _RPT, _RPT)])
        plsc.subcore_barrier()

        def half_step(j, rows_v, sem):
            pltpu.make_async_copy(h_hbm.at[src_v.at[j]], rows_v, sem).wait()
            pltpu.sync_copy(rows_v, acc_sh.at[dst_v.at[j]], add=True)

            @pl.when(j + 2 < _IGRP)
            def _():
                pltpu.async_copy(h_hbm.at[src_v.at[j + 2]], rows_v, sem)

        def group(g, carry):
            pltpu.sync_copy(src_hbm.at[wid].at[pl.ds(g * _IGRP, _IGRP)],
                            src_v)
            pltpu.sync_copy(dst_hbm.at[wid].at[pl.ds(g * _IGRP, _IGRP)],
                            dst_v)
            pltpu.async_copy(h_hbm.at[src_v.at[0]], rows_a, sem_a)
            pltpu.async_copy(h_hbm.at[src_v.at[1]], rows_b, sem_b)

            def step(i, carry2):
                half_step(2 * i, rows_a, sem_a)
                half_step(2 * i + 1, rows_b, sem_b)
                return carry2

            lax.fori_loop(0, _IGRP // 2, step, 0)
            return carry

        lax.fori_loop(0, _NCHUNK // _IGRP, group, 0)
        plsc.subcore_barrier()
        pltpu.sync_copy(acc_sh.at[pl.ds(sid * _RPT, _RPT)],
                        out_hbm.at[cid].at[pl.ds(sid * _RPT, _RPT)])

    return agg


_BLK = 1000


def _dinv_of(deg_ref):
    deg = deg_ref[0, :, 0] + deg_ref[1, :, 0] + 1.0
    return lax.rsqrt(jnp.maximum(deg, 1.0))


def _mm1_body(deg_ref, x_ref, w_ref, o_ref):
    dinv = _dinv_of(deg_ref)
    o_ref[...] = (x_ref[...] @ w_ref[...]) * dinv[:, None]


def _mm2_body(deg_ref, p_ref, hp_ref, b_ref, w_ref, o_ref):
    dinv = _dinv_of(deg_ref)
    agg = p_ref[0] + p_ref[1] + hp_ref[...]
    h = jnp.maximum(agg * dinv[:, None] + b_ref[...], 0.0)
    res = (h @ w_ref[...]) * dinv[:, None]
    o_ref[...] = jnp.concatenate([res, jnp.zeros_like(res)], axis=1)


def _mm3_body(deg_ref, p_ref, hp_ref, b_ref, w_ref, bfc_ref, o_ref):
    dinv = _dinv_of(deg_ref)
    agg = (p_ref[0] + p_ref[1] + hp_ref[...])[:, : _H // 2]
    h = jnp.maximum(agg * dinv[:, None] + b_ref[...], 0.0)
    logits = h @ w_ref[...] + bfc_ref[...]
    m = jnp.max(logits, axis=1, keepdims=True)
    lse = m + jnp.log(jnp.sum(jnp.exp(logits - m), axis=1, keepdims=True))
    o_ref[...] = logits - lse


def _row_block(nd):
    return pl.BlockSpec((_BLK, nd), lambda i: (i, 0))


def _deg_block():
    return pl.BlockSpec((2, _BLK, _DW), lambda i: (0, i, 0))


def _part_block(nd):
    return pl.BlockSpec((2, _BLK, nd), lambda i: (0, i, 0))


def _full(shape):
    return pl.BlockSpec(shape, lambda i: tuple(0 for _ in shape))


def kernel(x, edge_index, W1, b1, W2, b2, Wfc, bfc):
    src = edge_index[0]
    dst = edge_index[1]
    npad = _EPAD - _E
    src3 = jnp.concatenate(
        [src, jnp.zeros((npad,), jnp.int32)]).reshape(_NW, _NCHUNK, _CH)
    dst3 = jnp.concatenate(
        [dst, jnp.full((npad,), _N, jnp.int32)]).reshape(_NW, _NCHUNK, _CH)

    ones8 = jnp.ones((_CH, _DW), jnp.float32)
    z8 = jnp.zeros((_NPAD, _DW), jnp.float32)
    z128 = jnp.zeros((_NPAD, _H), jnp.float32)

    degp = _make_deg_kernel()(dst3, ones8, z8)[:, :_N, :]

    grid = (_N // _BLK,)
    h1p = pl.pallas_call(
        _mm1_body,
        grid=grid,
        in_specs=[_deg_block(), _row_block(_D), _full((_D, _H))],
        out_specs=_row_block(_H),
        out_shape=jax.ShapeDtypeStruct((_N, _H), jnp.float32),
    )(degp, x, W1)

    p1 = _make_agg_kernel(_H)(h1p, src3, dst3, z128)[:, :_N, :]

    h2p = pl.pallas_call(
        _mm2_body,
        grid=grid,
        in_specs=[_deg_block(), _part_block(_H), _row_block(_H),
                  _full((1, _H)), _full((_H, _H // 2))],
        out_specs=_row_block(_H),
        out_shape=jax.ShapeDtypeStruct((_N, _H), jnp.float32),
    )(degp, p1, h1p, b1.reshape(1, _H), W2)

    p2 = _make_agg_kernel(_H)(h2p, src3, dst3, z128)[:, :_N, :]

    out = pl.pallas_call(
        _mm3_body,
        grid=grid,
        in_specs=[_deg_block(), _part_block(_H), _row_block(_H),
                  _full((1, _H // 2)), _full((_H // 2, _C)), _full((1, _C))],
        out_specs=_row_block(_C),
        out_shape=jax.ShapeDtypeStruct((_N, _C), jnp.float32),
    )(degp, p2, h2p, b2.reshape(1, _H // 2), Wfc, bfc.reshape(1, _C))

    return out
